# Initial kernel scaffold; baseline (speedup 1.0000x reference)
import jax
import jax.numpy as jnp
from jax.experimental import pallas as pl


def kernel(VS, CC, Level, Depart, W_ih, W_hh, b_ih, b_hh, W_fc, b_fc):
    raise NotImplementedError("write your pallas kernel here")


# speedup vs baseline: 19.6634x; 19.6634x over previous
import jax
import jax.numpy as jnp
from jax.experimental import pallas as pl
from jax.experimental.pallas import tpu as pltpu

B = 4096
C = 28
K = 10
LP = 8
CK = C * K
H = 64
G4 = 4 * H
OUT = 768
BLK = 512
NB = B // BLK


def _col_selector():
    c_iota = jax.lax.broadcasted_iota(jnp.int32, (C, CK), 0)
    j_iota = jax.lax.broadcasted_iota(jnp.int32, (C, CK), 1)
    return jnp.where(c_iota == j_iota // K, 1.0, 0.0)


def _one_hots(vals, lev):
    n = vals.shape[0]
    l_iota = jax.lax.broadcasted_iota(jnp.int32, (n, LP), 1).astype(jnp.float32)
    lab_oh = jnp.where(lev == l_iota, 1.0, 0.0)
    R = _col_selector()
    vals_rep = jax.lax.dot_general(vals, R, (((1,), (0,)), ((), ())),
                                   preferred_element_type=jnp.float32)
    k_iota = (jax.lax.broadcasted_iota(jnp.int32, (n, CK), 1) % K).astype(jnp.float32)
    val_oh = jnp.where(vals_rep == k_iota, 1.0, 0.0)
    return lab_oh, val_oh, R


def _stats_kernel(vs_ref, lev_ref, counts_ref):
    i = pl.program_id(0)
    lab_oh, val_oh, _ = _one_hots(vs_ref[...], lev_ref[...])
    contrib = jax.lax.dot_general(lab_oh, val_oh, (((0,), (0,)), ((), ())),
                                  preferred_element_type=jnp.float32)

    @pl.when(i == 0)
    def _():
        counts_ref[...] = jnp.zeros_like(counts_ref)

    counts_ref[...] += contrib


def _main_kernel(counts_ref, vs_ref, lev_ref, wih_ref, bih_ref, bhh_ref,
                 wfc_ref, bfc_ref, out_ref):
    counts = counts_ref[...]
    nper = jnp.sum(counts[:, 0:K], axis=1, keepdims=True)
    ptab = counts / jnp.maximum(nper, 1.0)

    vals = vs_ref[...]
    lab_oh, val_oh, R = _one_hots(vals, lev_ref[...])
    tsel = jax.lax.dot_general(lab_oh, ptab, (((1,), (0,)), ((), ())),
                               preferred_element_type=jnp.float32)
    a = val_oh * tsel
    prob = jax.lax.dot_general(a, R, (((1,), (1,)), ((), ())),
                               preferred_element_type=jnp.float32)
    x = vals * prob

    gates = jax.lax.dot_general(x, wih_ref[...], (((1,), (1,)), ((), ())),
                                preferred_element_type=jnp.float32)
    gates = gates + bih_ref[...] + bhh_ref[...]
    i_g = jax.nn.sigmoid(gates[:, 0:H])
    g_g = jnp.tanh(gates[:, 2 * H:3 * H])
    o_g = jax.nn.sigmoid(gates[:, 3 * H:4 * H])
    h = o_g * jnp.tanh(i_g * g_g)
    out_ref[...] = jax.lax.dot_general(h, wfc_ref[...], (((1,), (1,)), ((), ())),
                                       preferred_element_type=jnp.float32) + bfc_ref[...]


def kernel(VS, CC, Level, Depart, W_ih, W_hh, b_ih, b_hh, W_fc, b_fc):
    lev_f = Level.astype(jnp.float32).reshape(B, 1)
    counts = pl.pallas_call(
        _stats_kernel,
        grid=(NB,),
        in_specs=[pl.BlockSpec((BLK, C), lambda i: (i, 0)),
                  pl.BlockSpec((BLK, 1), lambda i: (i, 0))],
        out_specs=pl.BlockSpec((LP, CK), lambda i: (0, 0)),
        out_shape=jax.ShapeDtypeStruct((LP, CK), jnp.float32),
    )(VS, lev_f)

    vs_feat = pl.pallas_call(
        _main_kernel,
        grid=(NB,),
        in_specs=[pl.BlockSpec((LP, CK), lambda i: (0, 0)),
                  pl.BlockSpec((BLK, C), lambda i: (i, 0)),
                  pl.BlockSpec((BLK, 1), lambda i: (i, 0)),
                  pl.BlockSpec((G4, C), lambda i: (0, 0)),
                  pl.BlockSpec((1, G4), lambda i: (0, 0)),
                  pl.BlockSpec((1, G4), lambda i: (0, 0)),
                  pl.BlockSpec((OUT, H), lambda i: (0, 0)),
                  pl.BlockSpec((1, OUT), lambda i: (0, 0))],
        out_specs=pl.BlockSpec((BLK, OUT), lambda i: (i, 0)),
        out_shape=jax.ShapeDtypeStruct((B, OUT), jnp.float32),
    )(counts, VS, lev_f, W_ih, b_ih.reshape(1, G4), b_hh.reshape(1, G4),
      W_fc, b_fc.reshape(1, OUT))

    return (jnp.squeeze(CC, axis=1), vs_feat, Level, Depart)
